# baseline (speedup 1.0000x reference)
import jax
import jax.numpy as jnp
from jax import lax
from jax.experimental import pallas as pl
from jax.experimental.pallas import tpu as pltpu
from jax.experimental.pallas import tpu_sc as plsc

B = 16384
K = 32
NC = 2
NS = 16
NW = NC * NS
BPW = B // NW
VCOLS = 1000000
TAIL0 = 999936
CHW = 512
NSC = 62
RANGE = NSC * CHW
HCAP = 768
PCAP = 64
TRASH = 32768
MDEPTH = 8
IROWS = 32776


def _p1_body(iall_hbm, ut_hbm, it_hbm, tu_hbm, ti_hbm, interm_hbm,
             iv, cu, ci, huj, hur, hij, hir, pj, pr, mini,
             tbu, tbi, sem_u, sem_i, sem_s):
    wid = lax.axis_index("s") * NC + lax.axis_index("c")
    lane = lax.iota(jnp.int32, 16)
    lo = wid * RANGE
    hi = lo + RANGE
    nvalid = jnp.minimum(NSC, lax.shift_right_logical(VCOLS - lo, 9))

    trash16 = lane * 0 + TRASH
    for d in range(MDEPTH):
        pltpu.async_copy(mini.at[d], interm_hbm.at[trash16], sem_s)

    def fire(c0, pb):
        pltpu.async_copy(ut_hbm.at[:, pl.ds(c0, CHW)], cu.at[pb],
                         sem_u.at[pb])
        pltpu.async_copy(it_hbm.at[:, pl.ds(c0, CHW)], ci.at[pb],
                         sem_i.at[pb])

    fire(lo, 0)

    pltpu.sync_copy(tu_hbm, tbu)
    pltpu.sync_copy(ti_hbm, tbi)

    offs = []
    for p in range(4):
        pltpu.sync_copy(iall_hbm.at[pl.ds(p * 8192, 8192)], iv)
        hj = huj if p < 2 else hij
        hr = hur if p < 2 else hir

        @pl.loop(0, 512, init_carry=(0 if p % 2 == 0 else offs[-1]))
        def _pre(v, off, p=p, hj=hj, hr=hr):
            vec = iv[pl.ds(v * 16, 16)]
            m = jnp.logical_and(vec >= lo, vec < hi)
            npc = plsc.all_reduce_population_count(m)[0]

            @pl.when(npc > 0)
            def _():
                plsc.store_compressed(hj.at[pl.ds(off, 16)],
                                      p * 8192 + v * 16 + lane, mask=m)
                plsc.store_compressed(hr.at[pl.ds(off, 16)], vec, mask=m)

            return off + npc

        offs.append(_pre)
    n_u = offs[1]
    n_i = offs[3]

    def compact_sub(hj, hr, nh, sc):
        nvec = lax.shift_right_logical(nh + 15, 4)

        @pl.loop(0, nvec, init_carry=0)
        def _cmp(v, np_sub):
            jv = hj[pl.ds(v * 16, 16)]
            rv = hr[pl.ds(v * 16, 16)]
            valid = (v * 16 + lane) < nh
            m = jnp.logical_and(
                valid, lax.shift_right_logical(rv - lo, 9) == sc)
            npc = plsc.all_reduce_population_count(m)[0]

            @pl.when(npc > 0)
            def _():
                plsc.store_compressed(pj.at[pl.ds(np_sub, 16)], jv, mask=m)
                plsc.store_compressed(pr.at[pl.ds(np_sub, 16)], rv, mask=m)

            return np_sub + npc

        return _cmp

    def extract_group(src, base, n, c0, pb, tail):
        jg = pj[pl.ds(base, 16)]
        rg = pr[pl.ds(base, 16)]
        jeff = jnp.where(lane < n, jg, lane * 0 + TRASH)
        rloc = jnp.bitwise_and(rg - c0, (64 if tail else CHW) - 1)
        pbv = lane * 0 + pb
        pltpu.make_async_copy(interm_hbm.at[pl.ds(IROWS - 16, 16)],
                              mini.at[0], sem_s).wait()
        for t in range(K):
            col = jnp.bitwise_and(lane + t, K - 1)
            if tail:
                uu = plsc.load_gather(src, [rloc, col])
            else:
                uu = plsc.load_gather(src, [col, rloc])
            plsc.store_scatter(mini, [pbv, lane, col], uu)
        pltpu.async_copy(mini.at[pb], interm_hbm.at[jeff], sem_s)

    def extract_all(src, np_sub, c0, pbmini, tail):
        ngrp = lax.shift_right_logical(np_sub + 15, 4)

        @pl.loop(0, ngrp, init_carry=pbmini)
        def _ext(g, pbm):
            extract_group(src, g * 16, np_sub - g * 16, c0, pbm, tail)
            return jnp.bitwise_and(pbm + 1, MDEPTH - 1)

        return _ext

    @pl.loop(0, nvalid, init_carry=0)
    def _main(sc, pbmini):
        c0 = lo + sc * CHW
        pb = jnp.bitwise_and(sc, 1)

        @pl.when(sc + 1 < nvalid)
        def _():
            fire(c0 + CHW, 1 - pb)

        pltpu.make_async_copy(ut_hbm.at[:, pl.ds(0, CHW)], cu.at[pb],
                              sem_u.at[pb]).wait()
        pltpu.make_async_copy(it_hbm.at[:, pl.ds(0, CHW)], ci.at[pb],
                              sem_i.at[pb]).wait()
        nu_sub = compact_sub(huj, hur, n_u, sc)
        carry = extract_all(cu.at[pb], nu_sub, c0, pbmini, False)
        ni_sub = compact_sub(hij, hir, n_i, sc)
        carry = extract_all(ci.at[pb], ni_sub, c0, carry, False)
        return carry

    @pl.when(wid == NW - 1)
    def _tail():
        nu_sub = compact_sub(huj, hur, n_u, 31)
        c1 = extract_all(tbu, nu_sub, TAIL0, _main, True)
        ni_sub = compact_sub(hij, hir, n_i, 31)
        extract_all(tbi, ni_sub, TAIL0, c1, True)

    for d in range(MDEPTH):
        pltpu.make_async_copy(interm_hbm.at[pl.ds(IROWS - 16, 16)],
                              mini.at[0], sem_s).wait()


def _p2_body(interm_hbm, du_hbm, di_hbm, ub_hbm, ib_hbm, out_hbm,
             idx_u, idx_i, vbu, vbi, ubv, ibv, outv, sem_b, sem_c):
    wid = lax.axis_index("s") * NC + lax.axis_index("c")
    lane = lax.iota(jnp.int32, 16)

    pltpu.sync_copy(du_hbm.at[wid], idx_u)
    pltpu.sync_copy(di_hbm.at[wid], idx_i)
    bias_copies = []
    for c in range(4):
        bias_copies.append(
            pltpu.async_copy(ub_hbm.at[idx_u.at[c]], ubv.at[c], sem_b))
        bias_copies.append(
            pltpu.async_copy(ib_hbm.at[idx_i.at[c]], ibv.at[c], sem_b))

    for c in range(4):
        r0 = wid * BPW + c * 128
        pltpu.async_copy(interm_hbm.at[pl.ds(r0, 128)], vbu, sem_c).wait()
        pltpu.async_copy(interm_hbm.at[pl.ds(B + r0, 128)], vbi,
                         sem_c).wait()
        if c == 0:
            for cp in bias_copies:
                cp.wait()
        for g in range(8):
            s = pl.ds(g * 16, 16)
            acc = ubv[c, s] + ibv[c, s]
            row = g * 16 + lane
            for t in range(K):
                col = jnp.bitwise_and(lane + t, K - 1)
                uu = plsc.load_gather(vbu, [row, col])
                ii = plsc.load_gather(vbi, [row, col])
                acc = acc + uu * ii
            outv[pl.ds(c * 128 + g * 16, 16)] = acc

    pltpu.sync_copy(outv, out_hbm.at[pl.ds(wid * BPW, BPW)])


@jax.jit
def _mf_stream(iall, du, di, ut, it, tu, ti, ub, ib):
    mesh = plsc.VectorSubcoreMesh(core_axis_name="c", subcore_axis_name="s")
    cparams = pltpu.CompilerParams(
        needs_layout_passes=False, use_tc_tiling_on_sc=True)
    interm = pl.kernel(
        _p1_body,
        out_type=jax.ShapeDtypeStruct((IROWS, 128), jnp.float32),
        mesh=mesh,
        compiler_params=cparams,
        scratch_types=[
            pltpu.VMEM((8192,), jnp.int32),
            pltpu.VMEM((2, K, CHW), jnp.float32),
            pltpu.VMEM((2, K, CHW), jnp.float32),
            pltpu.VMEM((HCAP,), jnp.int32),
            pltpu.VMEM((HCAP,), jnp.int32),
            pltpu.VMEM((HCAP,), jnp.int32),
            pltpu.VMEM((HCAP,), jnp.int32),
            pltpu.VMEM((PCAP,), jnp.int32),
            pltpu.VMEM((PCAP,), jnp.int32),
            pltpu.VMEM((MDEPTH, 16, 128), jnp.float32),
            pltpu.VMEM((64, 128), jnp.float32),
            pltpu.VMEM((64, 128), jnp.float32),
            pltpu.SemaphoreType.DMA((2,)),
            pltpu.SemaphoreType.DMA((2,)),
            pltpu.SemaphoreType.DMA,
        ],
    )(iall, ut, it, tu, ti)
    return pl.kernel(
        _p2_body,
        out_type=jax.ShapeDtypeStruct((B,), jnp.float32),
        mesh=mesh,
        compiler_params=cparams,
        scratch_types=[
            pltpu.VMEM((4, 128), jnp.int32),
            pltpu.VMEM((4, 128), jnp.int32),
            pltpu.VMEM((128, 128), jnp.float32),
            pltpu.VMEM((128, 128), jnp.float32),
            pltpu.VMEM((4, 128), jnp.float32),
            pltpu.VMEM((4, 128), jnp.float32),
            pltpu.VMEM((BPW,), jnp.float32),
            pltpu.SemaphoreType.DMA,
            pltpu.SemaphoreType.DMA,
        ],
    )(interm, du, di, ub, ib)


def kernel(data_u, data_i, u_emb, i_emb, user_b, item_b):
    du32 = data_u.astype(jnp.int32)
    di32 = data_i.astype(jnp.int32)
    iall = jnp.concatenate([du32, di32])
    du = du32.reshape(NW, 4, 128)
    di = di32.reshape(NW, 4, 128)
    ut = u_emb.T
    it = i_emb.T
    tu = jnp.pad(u_emb[TAIL0:, :], ((0, 0), (0, 128 - K)))
    ti = jnp.pad(i_emb[TAIL0:, :], ((0, 0), (0, 128 - K)))
    ub = user_b.reshape(-1)
    ib = item_b.reshape(-1)
    return _mf_stream(iall, du, di, ut, it, tu, ti, ub, ib)


# speedup vs baseline: 1.0008x; 1.0008x over previous
import jax
import jax.numpy as jnp
from jax import lax
from jax.experimental import pallas as pl
from jax.experimental.pallas import tpu as pltpu
from jax.experimental.pallas import tpu_sc as plsc

B = 16384
K = 32
NC = 2
NS = 16
NW = NC * NS
BPW = B // NW
VCOLS = 1000000
TAIL0 = 999936
CHW = 512
NSC = 62
RANGE = NSC * CHW
HCAP = 768
PCAP = 64
TRASH = 32768
MDEPTH = 8
IROWS = 32776


def _p1_body(iall_hbm, ut_hbm, it_hbm, tu_hbm, ti_hbm, interm_hbm,
             iv, cu, ci, huj, hur, hij, hir, pj, pr, mini,
             tbu, tbi, sem_u, sem_i, sem_s):
    wid = lax.axis_index("s") * NC + lax.axis_index("c")
    lane = lax.iota(jnp.int32, 16)
    lo = wid * RANGE
    hi = lo + RANGE
    nvalid = jnp.minimum(NSC, lax.shift_right_logical(VCOLS - lo, 9))

    trash16 = lane * 0 + TRASH
    for d in range(MDEPTH):
        pltpu.async_copy(mini.at[d], interm_hbm.at[trash16], sem_s)

    def fire(c0, pb):
        for g in range(4):
            rs = pl.ds(g * 8, 8)
            pltpu.async_copy(ut_hbm.at[rs, pl.ds(c0, CHW)],
                             cu.at[pb, rs], sem_u.at[pb])
            pltpu.async_copy(it_hbm.at[rs, pl.ds(c0, CHW)],
                             ci.at[pb, rs], sem_i.at[pb])

    fire(lo, 0)

    pltpu.sync_copy(tu_hbm, tbu)
    pltpu.sync_copy(ti_hbm, tbi)

    offs = []
    for p in range(4):
        pltpu.sync_copy(iall_hbm.at[pl.ds(p * 8192, 8192)], iv)
        hj = huj if p < 2 else hij
        hr = hur if p < 2 else hir

        @pl.loop(0, 512, init_carry=(0 if p % 2 == 0 else offs[-1]))
        def _pre(v, off, p=p, hj=hj, hr=hr):
            vec = iv[pl.ds(v * 16, 16)]
            m = jnp.logical_and(vec >= lo, vec < hi)
            npc = plsc.all_reduce_population_count(m)[0]

            @pl.when(npc > 0)
            def _():
                plsc.store_compressed(hj.at[pl.ds(off, 16)],
                                      p * 8192 + v * 16 + lane, mask=m)
                plsc.store_compressed(hr.at[pl.ds(off, 16)], vec, mask=m)

            return off + npc

        offs.append(_pre)
    n_u = offs[1]
    n_i = offs[3]

    def compact_sub(hj, hr, nh, sc):
        nvec = lax.shift_right_logical(nh + 15, 4)

        @pl.loop(0, nvec, init_carry=0)
        def _cmp(v, np_sub):
            jv = hj[pl.ds(v * 16, 16)]
            rv = hr[pl.ds(v * 16, 16)]
            valid = (v * 16 + lane) < nh
            m = jnp.logical_and(
                valid, lax.shift_right_logical(rv - lo, 9) == sc)
            npc = plsc.all_reduce_population_count(m)[0]

            @pl.when(npc > 0)
            def _():
                plsc.store_compressed(pj.at[pl.ds(np_sub, 16)], jv, mask=m)
                plsc.store_compressed(pr.at[pl.ds(np_sub, 16)], rv, mask=m)

            return np_sub + npc

        return _cmp

    def extract_group(src, base, n, c0, pb, tail):
        jg = pj[pl.ds(base, 16)]
        rg = pr[pl.ds(base, 16)]
        jeff = jnp.where(lane < n, jg, lane * 0 + TRASH)
        rloc = jnp.bitwise_and(rg - c0, (64 if tail else CHW) - 1)
        pbv = lane * 0 + pb
        pltpu.make_async_copy(interm_hbm.at[pl.ds(IROWS - 16, 16)],
                              mini.at[0], sem_s).wait()
        for t in range(K):
            col = jnp.bitwise_and(lane + t, K - 1)
            if tail:
                uu = plsc.load_gather(src, [rloc, col])
            else:
                uu = plsc.load_gather(src, [col, rloc])
            plsc.store_scatter(mini, [pbv, lane, col], uu)
        pltpu.async_copy(mini.at[pb], interm_hbm.at[jeff], sem_s)

    def extract_all(src, np_sub, c0, pbmini, tail):
        ngrp = lax.shift_right_logical(np_sub + 15, 4)

        @pl.loop(0, ngrp, init_carry=pbmini)
        def _ext(g, pbm):
            extract_group(src, g * 16, np_sub - g * 16, c0, pbm, tail)
            return jnp.bitwise_and(pbm + 1, MDEPTH - 1)

        return _ext

    @pl.loop(0, nvalid, init_carry=0)
    def _main(sc, pbmini):
        c0 = lo + sc * CHW
        pb = jnp.bitwise_and(sc, 1)

        @pl.when(sc + 1 < nvalid)
        def _():
            fire(c0 + CHW, 1 - pb)

        pltpu.make_async_copy(ut_hbm.at[:, pl.ds(0, CHW)], cu.at[pb],
                              sem_u.at[pb]).wait()
        pltpu.make_async_copy(it_hbm.at[:, pl.ds(0, CHW)], ci.at[pb],
                              sem_i.at[pb]).wait()
        nu_sub = compact_sub(huj, hur, n_u, sc)
        carry = extract_all(cu.at[pb], nu_sub, c0, pbmini, False)
        ni_sub = compact_sub(hij, hir, n_i, sc)
        carry = extract_all(ci.at[pb], ni_sub, c0, carry, False)
        return carry

    @pl.when(wid == NW - 1)
    def _tail():
        nu_sub = compact_sub(huj, hur, n_u, 31)
        c1 = extract_all(tbu, nu_sub, TAIL0, _main, True)
        ni_sub = compact_sub(hij, hir, n_i, 31)
        extract_all(tbi, ni_sub, TAIL0, c1, True)

    for d in range(MDEPTH):
        pltpu.make_async_copy(interm_hbm.at[pl.ds(IROWS - 16, 16)],
                              mini.at[0], sem_s).wait()


def _p2_body(interm_hbm, du_hbm, di_hbm, ub_hbm, ib_hbm, out_hbm,
             idx_u, idx_i, vbu, vbi, ubv, ibv, outv, sem_b, sem_c):
    wid = lax.axis_index("s") * NC + lax.axis_index("c")
    lane = lax.iota(jnp.int32, 16)

    pltpu.sync_copy(du_hbm.at[wid], idx_u)
    pltpu.sync_copy(di_hbm.at[wid], idx_i)
    bias_copies = []
    for c in range(4):
        bias_copies.append(
            pltpu.async_copy(ub_hbm.at[idx_u.at[c]], ubv.at[c], sem_b))
        bias_copies.append(
            pltpu.async_copy(ib_hbm.at[idx_i.at[c]], ibv.at[c], sem_b))

    for c in range(4):
        r0 = wid * BPW + c * 128
        pltpu.async_copy(interm_hbm.at[pl.ds(r0, 128)], vbu, sem_c).wait()
        pltpu.async_copy(interm_hbm.at[pl.ds(B + r0, 128)], vbi,
                         sem_c).wait()
        if c == 0:
            for cp in bias_copies:
                cp.wait()
        for g in range(8):
            s = pl.ds(g * 16, 16)
            acc = ubv[c, s] + ibv[c, s]
            row = g * 16 + lane
            for t in range(K):
                col = jnp.bitwise_and(lane + t, K - 1)
                uu = plsc.load_gather(vbu, [row, col])
                ii = plsc.load_gather(vbi, [row, col])
                acc = acc + uu * ii
            outv[pl.ds(c * 128 + g * 16, 16)] = acc

    pltpu.sync_copy(outv, out_hbm.at[pl.ds(wid * BPW, BPW)])


@jax.jit
def _mf_stream(iall, du, di, ut, it, tu, ti, ub, ib):
    mesh = plsc.VectorSubcoreMesh(core_axis_name="c", subcore_axis_name="s")
    cparams = pltpu.CompilerParams(
        needs_layout_passes=False, use_tc_tiling_on_sc=True)
    interm = pl.kernel(
        _p1_body,
        out_type=jax.ShapeDtypeStruct((IROWS, 128), jnp.float32),
        mesh=mesh,
        compiler_params=cparams,
        scratch_types=[
            pltpu.VMEM((8192,), jnp.int32),
            pltpu.VMEM((2, K, CHW), jnp.float32),
            pltpu.VMEM((2, K, CHW), jnp.float32),
            pltpu.VMEM((HCAP,), jnp.int32),
            pltpu.VMEM((HCAP,), jnp.int32),
            pltpu.VMEM((HCAP,), jnp.int32),
            pltpu.VMEM((HCAP,), jnp.int32),
            pltpu.VMEM((PCAP,), jnp.int32),
            pltpu.VMEM((PCAP,), jnp.int32),
            pltpu.VMEM((MDEPTH, 16, 128), jnp.float32),
            pltpu.VMEM((64, 128), jnp.float32),
            pltpu.VMEM((64, 128), jnp.float32),
            pltpu.SemaphoreType.DMA((2,)),
            pltpu.SemaphoreType.DMA((2,)),
            pltpu.SemaphoreType.DMA,
        ],
    )(iall, ut, it, tu, ti)
    return pl.kernel(
        _p2_body,
        out_type=jax.ShapeDtypeStruct((B,), jnp.float32),
        mesh=mesh,
        compiler_params=cparams,
        scratch_types=[
            pltpu.VMEM((4, 128), jnp.int32),
            pltpu.VMEM((4, 128), jnp.int32),
            pltpu.VMEM((128, 128), jnp.float32),
            pltpu.VMEM((128, 128), jnp.float32),
            pltpu.VMEM((4, 128), jnp.float32),
            pltpu.VMEM((4, 128), jnp.float32),
            pltpu.VMEM((BPW,), jnp.float32),
            pltpu.SemaphoreType.DMA,
            pltpu.SemaphoreType.DMA,
        ],
    )(interm, du, di, ub, ib)


def kernel(data_u, data_i, u_emb, i_emb, user_b, item_b):
    du32 = data_u.astype(jnp.int32)
    di32 = data_i.astype(jnp.int32)
    iall = jnp.concatenate([du32, di32])
    du = du32.reshape(NW, 4, 128)
    di = di32.reshape(NW, 4, 128)
    ut = u_emb.T
    it = i_emb.T
    tu = jnp.pad(u_emb[TAIL0:, :], ((0, 0), (0, 128 - K)))
    ti = jnp.pad(i_emb[TAIL0:, :], ((0, 0), (0, 128 - K)))
    ub = user_b.reshape(-1)
    ib = item_b.reshape(-1)
    return _mf_stream(iall, du, di, ut, it, tu, ti, ub, ib)


# speedup vs baseline: 1.9001x; 1.8985x over previous
import jax
import jax.numpy as jnp
from jax import lax
from jax.experimental import pallas as pl
from jax.experimental.pallas import tpu as pltpu
from jax.experimental.pallas import tpu_sc as plsc

B = 16384
K = 32
NC = 2
NS = 16
NW = NC * NS
BPW = B // NW
CH = 128
NCH = BPW // CH
PAD = 17


def _mf_body(du_hbm, di_hbm, ue_hbm, ie_hbm, ub_hbm, ib_hbm, out_hbm,
             idx_u, idx_i, urows, irows, ubv, ibv, spad, outv, sem):
    wid = lax.axis_index("s") * NC + lax.axis_index("c")

    pltpu.sync_copy(du_hbm.at[wid], idx_u)
    pltpu.sync_copy(di_hbm.at[wid], idx_i)

    copies = []
    for c in range(NCH):
        copies.append(pltpu.async_copy(ue_hbm.at[idx_u.at[c]], urows.at[c], sem))
        copies.append(pltpu.async_copy(ie_hbm.at[idx_i.at[c]], irows.at[c], sem))
        copies.append(pltpu.async_copy(ub_hbm.at[idx_u.at[c]], ubv.at[c], sem))
        copies.append(pltpu.async_copy(ib_hbm.at[idx_i.at[c]], ibv.at[c], sem))
    for cp in copies:
        cp.wait()

    lane = jnp.arange(16, dtype=jnp.int32)
    for c in range(NCH):
        @pl.loop(0, CH, unroll=4)
        def _(rr):
            u0 = urows[c, rr, pl.ds(0, 16)]
            u1 = urows[c, rr, pl.ds(16, 16)]
            i0 = irows[c, rr, pl.ds(0, 16)]
            i1 = irows[c, rr, pl.ds(16, 16)]
            t = u0 * i0 + u1 * i1
            plsc.store_scatter(spad, [(c * CH + rr) * PAD + lane], t)

    gpc = CH // 16
    for g in range(BPW // 16):
        base = (g * 16 + lane) * PAD
        acc = (ubv[g // gpc, pl.ds((g % gpc) * 16, 16)] +
               ibv[g // gpc, pl.ds((g % gpc) * 16, 16)])
        for j in range(16):
            acc = acc + plsc.load_gather(spad, [base + j])
        outv[pl.ds(g * 16, 16)] = acc

    pltpu.sync_copy(outv, out_hbm.at[pl.ds(wid * BPW, BPW)])


@jax.jit
def _mf(du, di, u_emb, i_emb, ub, ib):
    mesh = plsc.VectorSubcoreMesh(core_axis_name="c", subcore_axis_name="s")
    return pl.kernel(
        _mf_body,
        out_type=jax.ShapeDtypeStruct((B,), jnp.float32),
        mesh=mesh,
        compiler_params=pltpu.CompilerParams(
            needs_layout_passes=False, use_tc_tiling_on_sc=False),
        scratch_types=[
            pltpu.VMEM((NCH, CH), jnp.int32),
            pltpu.VMEM((NCH, CH), jnp.int32),
            pltpu.VMEM((NCH, CH, K), jnp.float32),
            pltpu.VMEM((NCH, CH, K), jnp.float32),
            pltpu.VMEM((NCH, CH), jnp.float32),
            pltpu.VMEM((NCH, CH), jnp.float32),
            pltpu.VMEM((BPW * PAD,), jnp.float32),
            pltpu.VMEM((BPW,), jnp.float32),
            pltpu.SemaphoreType.DMA,
        ],
    )(du, di, u_emb, i_emb, ub, ib)


def kernel(data_u, data_i, u_emb, i_emb, user_b, item_b):
    du = data_u.astype(jnp.int32).reshape(NW, NCH, CH)
    di = data_i.astype(jnp.int32).reshape(NW, NCH, CH)
    ub = user_b.reshape(-1)
    ib = item_b.reshape(-1)
    return _mf(du, di, u_emb, i_emb, ub, ib)
